# baseline (speedup 1.0000x reference)
import functools

import jax
import jax.numpy as jnp
from jax import lax
from jax.experimental import pallas as pl
from jax.experimental.pallas import tpu as pltpu
from jax.experimental.pallas import tpu_sc as plsc

NC = 2
NS = 16
NW = NC * NS

D = 768
CHUNK = 40
LANES = 16


def _body(x_ref, tok_ref, pos_ref, out_ref, idx_v, buf, pos_v, sem):
    cid = lax.axis_index("c")
    sid = lax.axis_index("s")
    wid = sid * NC + cid
    wbase = pl.multiple_of(wid * 6400, 6400)

    for pb in range(200 // CHUNK):
        pltpu.sync_copy(pos_ref.at[pl.ds(pb * CHUNK, CHUNK)], pos_v)

        def seq_body(seq, _, pb=pb):
            base = pl.multiple_of(wbase + seq * 200 + pb * CHUNK, CHUNK)
            pltpu.sync_copy(x_ref.at[pl.ds(base, CHUNK)], idx_v)
            pltpu.async_copy(tok_ref.at[idx_v], buf, sem).wait()

            def row_body(r, _):
                for c in range(D // LANES):
                    sl = pl.ds(LANES * c, LANES)
                    plsc.addupdate(buf.at[r, sl], pos_v[r, sl])
                return 0

            lax.fori_loop(0, CHUNK, row_body, 0, unroll=False)
            pltpu.sync_copy(buf, out_ref.at[pl.ds(base, CHUNK)])
            return 0

        lax.fori_loop(0, 32, seq_body, 0, unroll=False)


@jax.jit
def _embed(x_flat, tok_table, pos_table):
    n = x_flat.shape[0]
    mesh = plsc.VectorSubcoreMesh(core_axis_name="c", subcore_axis_name="s")
    k = pl.kernel(
        _body,
        out_type=jax.ShapeDtypeStruct((n, D), jnp.float32),
        mesh=mesh,
        scratch_types=[
            pltpu.VMEM((CHUNK,), jnp.int32),
            pltpu.VMEM((CHUNK, D), jnp.float32),
            pltpu.VMEM((CHUNK, D), jnp.float32),
            pltpu.SemaphoreType.DMA,
        ],
    )
    return k(x_flat, tok_table, pos_table)


def kernel(x, tok_table, pos_table):
    b, s = x.shape
    out = _embed(x.reshape(-1).astype(jnp.int32), tok_table, pos_table)
    return out.reshape(b, s, D)


# speedup vs baseline: 1.0564x; 1.0564x over previous
import jax
import jax.numpy as jnp
from jax import lax
from jax.experimental import pallas as pl
from jax.experimental.pallas import tpu as pltpu
from jax.experimental.pallas import tpu_sc as plsc

NC = 2
NS = 16
NW = NC * NS

D = 768
CHUNK = 40
LANES = 16
SEQS = 32
PHASES = 200 // CHUNK


def _add_pos(buf_b, pos_v):
    def row_body(r, _):
        for c in range(D // LANES):
            sl = pl.ds(LANES * c, LANES)
            plsc.addupdate(buf_b.at[r, sl], pos_v[r, sl])
        return 0

    lax.fori_loop(0, CHUNK, row_body, 0, unroll=False)


def _body(x_ref, tok_ref, pos_ref, out_ref, idxr, buf, pos_v, sem_g, sem_s):
    cid = lax.axis_index("c")
    sid = lax.axis_index("s")
    wid = sid * NC + cid
    wbase = pl.multiple_of(wid * (SEQS * 200), 200)

    def wait_gather(b):
        pltpu.make_async_copy(tok_ref.at[idxr.at[b]], buf.at[b],
                              sem_g.at[b]).wait()

    def wait_store(b):
        pltpu.make_async_copy(buf.at[b], out_ref.at[pl.ds(wbase, CHUNK)],
                              sem_s.at[b]).wait()

    for pb in range(PHASES):
        pltpu.sync_copy(pos_ref.at[pl.ds(pb * CHUNK, CHUNK)], pos_v)
        pbase = pb * CHUNK

        def gather_p(s, b, pbase=pbase):
            base = pl.multiple_of(wbase + s * 200 + pbase, CHUNK)
            pltpu.sync_copy(x_ref.at[pl.ds(base, CHUNK)], idxr.at[b])
            pltpu.async_copy(tok_ref.at[idxr.at[b]], buf.at[b], sem_g.at[b])

        def store_p(s, b, pbase=pbase):
            base = pl.multiple_of(wbase + s * 200 + pbase, CHUNK)
            pltpu.async_copy(buf.at[b], out_ref.at[pl.ds(base, CHUNK)],
                             sem_s.at[b])

        gather_p(0, 0)

        @pl.loop(0, SEQS // 2)
        def pair(i):
            s0 = i * 2

            @pl.when(i >= 1)
            def _():
                wait_store(1)
            gather_p(s0 + 1, 1)
            wait_gather(0)
            _add_pos(buf.at[0], pos_v)
            store_p(s0, 0)

            @pl.when(i <= SEQS // 2 - 2)
            def _():
                wait_store(0)
                gather_p(s0 + 2, 0)
            wait_gather(1)
            _add_pos(buf.at[1], pos_v)
            store_p(s0 + 1, 1)

        wait_store(0)
        wait_store(1)


@jax.jit
def _embed(x_flat, tok_table, pos_table):
    n = x_flat.shape[0]
    mesh = plsc.VectorSubcoreMesh(core_axis_name="c", subcore_axis_name="s")
    k = pl.kernel(
        _body,
        out_type=jax.ShapeDtypeStruct((n, D), jnp.float32),
        mesh=mesh,
        scratch_types=[
            pltpu.VMEM((2, CHUNK), jnp.int32),
            pltpu.VMEM((2, CHUNK, D), jnp.float32),
            pltpu.VMEM((CHUNK, D), jnp.float32),
            pltpu.SemaphoreType.DMA((2,)),
            pltpu.SemaphoreType.DMA((2,)),
        ],
    )
    return k(x_flat, tok_table, pos_table)


def kernel(x, tok_table, pos_table):
    b, s = x.shape
    out = _embed(x.reshape(-1).astype(jnp.int32), tok_table, pos_table)
    return out.reshape(b, s, D)
